# baseline (speedup 1.0000x reference)
import functools

import jax
import jax.numpy as jnp
from jax import lax
from jax.experimental import pallas as pl
from jax.experimental.pallas import tpu as pltpu
from jax.experimental.pallas import tpu_sc as plsc

EMBED = 64
LANES = 16
NC, NS = 2, 16
NW = NC * NS
CHUNK = 128
NBUF = 5
SCALE = 8.0


@jax.jit
def _embed_sc(tokens3, table):
    nw, nchunks, chunk = tokens3.shape
    n = nw * nchunks * chunk
    mesh = plsc.VectorSubcoreMesh(core_axis_name="c", subcore_axis_name="s")

    scratch = [pltpu.VMEM((nchunks, chunk), jnp.int32)]
    scratch += [pltpu.VMEM((chunk, EMBED), jnp.float32) for _ in range(2 * NBUF)]
    scratch += [pltpu.SemaphoreType.DMA for _ in range(2 * NBUF)]

    @functools.partial(
        pl.kernel,
        mesh=mesh,
        compiler_params=pltpu.CompilerParams(use_tc_tiling_on_sc=False),
        out_type=jax.ShapeDtypeStruct((n, EMBED), jnp.float32),
        scratch_types=scratch,
    )
    def k(tok_hbm, tab_hbm, out_hbm, *sc):
        idx_v = sc[0]
        gbuf = sc[1:1 + NBUF]
        obuf = sc[1 + NBUF:1 + 2 * NBUF]
        gsem = sc[1 + 2 * NBUF:1 + 3 * NBUF]
        osem = sc[1 + 3 * NBUF:1 + 4 * NBUF]
        wid = lax.axis_index("s") * NC + lax.axis_index("c")
        base = wid * (nchunks * chunk)
        pltpu.sync_copy(tok_hbm.at[wid], idx_v)
        for b in range(NBUF):
            pltpu.async_copy(tab_hbm.at[idx_v.at[b]], gbuf[b], gsem[b])

        def outer(g, carry):
            for b in range(NBUF):
                j = g * NBUF + b
                pltpu.make_async_copy(
                    tab_hbm.at[idx_v.at[j]], gbuf[b], gsem[b]).wait()

                @pl.when(g > 0)
                def _wait_out(b=b):
                    pltpu.make_async_copy(
                        obuf[b], out_hbm.at[pl.ds(0, chunk)], osem[b]).wait()

                def scale_body(r, c, b=b):
                    for kk in range(EMBED // LANES):
                        sl = pl.ds(kk * LANES, LANES)
                        obuf[b][r, sl] = gbuf[b][r, sl] * SCALE
                    return c

                lax.fori_loop(0, chunk, scale_body, 0, unroll=4)

                @pl.when(j + NBUF < nchunks)
                def _fire_next(b=b, j=j):
                    pltpu.async_copy(
                        tab_hbm.at[idx_v.at[j + NBUF]], gbuf[b], gsem[b])

                pltpu.async_copy(
                    obuf[b], out_hbm.at[pl.ds(base + j * chunk, chunk)],
                    osem[b])
            return carry

        lax.fori_loop(0, nchunks // NBUF, outer, 0)
        for b in range(NBUF):
            pltpu.make_async_copy(
                obuf[b], out_hbm.at[pl.ds(0, chunk)], osem[b]).wait()

    return k(tokens3, table)


def kernel(tokens, input_embedding_table):
    b, l = tokens.shape
    n = b * l
    tokens3 = tokens.reshape(NW, n // (NW * CHUNK), CHUNK).astype(jnp.int32)
    out = _embed_sc(tokens3, input_embedding_table)
    return out.reshape(b, l, EMBED)
